# baseline (speedup 1.0000x reference)
import functools

import jax
import jax.numpy as jnp
from jax import lax
from jax.experimental import pallas as pl
from jax.experimental.pallas import tpu as pltpu
from jax.experimental.pallas import tpu_sc as plsc

_VOCAB = 1000000
_EMB = 64
_BATCH = 16384
_HIST = 50

_HALF = _EMB // 2

_NC = 2
_NS = 16
_NW = _NC * _NS

_N = _BATCH * _HIST
_PER_W = _N // _NW
_G = 128
_NSTEP = _PER_W // _G
_R = 10
_L = 16

_PROWS = _VOCAB // _NW
_PC = 625
_PCHUNKS = _PROWS // _PC


def _rne_hi(bits):
  return (bits + 0x7FFF + ((bits >> 16) & 1)) & jnp.int32(-65536)


def _pack_body(w_hbm, pk_hbm, in_v, out_v, lsems, ssems):
  wid = lax.axis_index("s") * _NC + lax.axis_index("c")
  base = wid * _PROWS

  def load_start(c, b):
    pltpu.async_copy(
        w_hbm.at[pl.ds(base + c * _PC, _PC)], in_v.at[b], lsems[b])

  def load_wait(c, b):
    pltpu.make_async_copy(
        w_hbm.at[pl.ds(base + c * _PC, _PC)], in_v.at[b], lsems[b]).wait()

  def store_start(c, b):
    pltpu.async_copy(
        out_v.at[b], pk_hbm.at[pl.ds(base + c * _PC, _PC)], ssems[b])

  def store_wait(c, b):
    pltpu.make_async_copy(
        out_v.at[b], pk_hbm.at[pl.ds(base + c * _PC, _PC)], ssems[b]).wait()

  def compute(b):
    xi = in_v.at[b]
    xo = out_v.at[b]

    @pl.loop(0, _PC)
    def _rows(r):
      for k in range(2):
        lo = plsc.bitcast(xi[r, pl.ds(k * _L, _L)], jnp.int32)
        hi = plsc.bitcast(xi[r, pl.ds(_HALF + k * _L, _L)], jnp.int32)
        lo16 = jnp.right_shift(_rne_hi(lo), 16) & jnp.int32(0xFFFF)
        xo[r, pl.ds(k * _L, _L)] = lo16 | _rne_hi(hi)

  load_start(0, 0)

  @pl.loop(0, _PCHUNKS // 2)
  def _steps(i):
    for j in range(2):
      c = 2 * i + j
      b = j

      load_wait(c, b)

      @pl.when(c + 1 < _PCHUNKS)
      def _():
        load_start(c + 1, 1 - b)

      @pl.when(c >= 2)
      def _():
        store_wait(c - 2, b)

      compute(b)
      store_start(c, b)

  store_wait(_PCHUNKS - 2, 0)
  store_wait(_PCHUNKS - 1, 1)


def _emb_body(idx_hbm, table_hbm, out_hbm, idx_v, pk_v, fout_v, gsems, ssems):
  wid = lax.axis_index("s") * _NC + lax.axis_index("c")
  base = wid * _PER_W

  pltpu.sync_copy(idx_hbm.at[wid], idx_v)

  def gather_start(m, b):
    pltpu.async_copy(table_hbm.at[idx_v.at[m]], pk_v.at[b], gsems[b])

  def gather_wait(b):
    pltpu.make_async_copy(
        table_hbm.at[pl.ds(0, _G)], pk_v.at[b], gsems[b]).wait()

  def store_start(m, fb):
    pltpu.async_copy(
        fout_v.at[fb], out_hbm.at[pl.ds(base + m * _G, _G)], ssems[fb])

  def store_wait(m, fb):
    pltpu.make_async_copy(
        fout_v.at[fb], out_hbm.at[pl.ds(base + m * _G, _G)], ssems[fb]).wait()

  def unpack(b, fb):
    pk = pk_v.at[b]
    fo = fout_v.at[fb]

    @pl.loop(0, _G)
    def _rows(r):
      for k in range(2):
        w = pk[r, pl.ds(k * _L, _L)]
        fo[r, pl.ds(k * _L, _L)] = plsc.bitcast(w << 16, jnp.float32)
        fo[r, pl.ds(_HALF + k * _L, _L)] = plsc.bitcast(
            w & jnp.int32(-65536), jnp.float32)

  for j in range(_R - 1):
    gather_start(j, j)

  @pl.loop(0, _NSTEP // _R)
  def _steps(i):
    for j in range(_R):
      m = _R * i + j
      b = j
      bp = (j - 1) % _R
      fb = j % 2

      @pl.when(m + _R - 1 < _NSTEP)
      def _():
        gather_start(m + _R - 1, bp)

      gather_wait(b)

      @pl.when(m >= 2)
      def _():
        store_wait(m - 2, fb)

      unpack(b, fb)
      store_start(m, fb)

  store_wait(_NSTEP - 2, _NSTEP % 2)
  store_wait(_NSTEP - 1, (_NSTEP - 1) % 2)


@jax.jit
def _emb(idx, W):
  mesh = plsc.VectorSubcoreMesh(
      core_axis_name="c", subcore_axis_name="s",
      num_cores=_NC, num_subcores=_NS)
  params = pltpu.CompilerParams(
      use_tc_tiling_on_sc=False, needs_layout_passes=False)

  pack = functools.partial(
      pl.kernel,
      mesh=mesh,
      out_type=jax.ShapeDtypeStruct((_VOCAB, _HALF), jnp.int32),
      scratch_types=[
          pltpu.VMEM((2, _PC, _EMB), jnp.float32),
          pltpu.VMEM((2, _PC, _HALF), jnp.int32),
          [pltpu.SemaphoreType.DMA] * 2,
          [pltpu.SemaphoreType.DMA] * 2,
      ],
      compiler_params=params,
  )(_pack_body)

  gather = functools.partial(
      pl.kernel,
      mesh=mesh,
      out_type=jax.ShapeDtypeStruct((_N, _EMB), jnp.float32),
      scratch_types=[
          pltpu.VMEM((_NSTEP, _G), jnp.int32),
          pltpu.VMEM((_R, _G, _HALF), jnp.int32),
          pltpu.VMEM((2, _G, _EMB), jnp.float32),
          [pltpu.SemaphoreType.DMA] * _R,
          [pltpu.SemaphoreType.DMA] * 2,
      ],
      compiler_params=params,
  )(_emb_body)

  return gather(idx, pack(W))


def kernel(data, W):
  idx = data.reshape(_NW, _NSTEP, _G)
  out = _emb(idx, W)
  return out.reshape(_BATCH, _HIST, _EMB)


# speedup vs baseline: 1.4554x; 1.4554x over previous
import functools

import jax
import jax.numpy as jnp
from jax import lax
from jax.experimental import pallas as pl
from jax.experimental.pallas import tpu as pltpu
from jax.experimental.pallas import tpu_sc as plsc

_VOCAB = 1000000
_EMB = 64
_BATCH = 16384
_HIST = 50

_NC = 2
_NS = 16
_NW = _NC * _NS

_PER_W = _BATCH // _NW
_R = 8


def _emb_body(idx_hbm, table_hbm, out_hbm, idx_v, rows_v, gsems, ssems):
  wid = lax.axis_index("s") * _NC + lax.axis_index("c")
  base = wid * _PER_W

  pltpu.sync_copy(idx_hbm.at[wid], idx_v)

  def gather_start(m, b):
    pltpu.async_copy(table_hbm.at[idx_v.at[m]], rows_v.at[b], gsems[b])

  def gather_wait(b):
    pltpu.make_async_copy(
        table_hbm.at[pl.ds(0, _HIST)], rows_v.at[b], gsems[b]).wait()

  def store_start(m, b):
    pltpu.async_copy(rows_v.at[b], out_hbm.at[base + m], ssems[b])

  def store_wait(m, b):
    pltpu.make_async_copy(
        rows_v.at[b], out_hbm.at[base + m], ssems[b]).wait()

  for j in range(_R - 1):
    gather_start(j, j)

  @pl.loop(0, _PER_W // _R)
  def _steps(i):
    for j in range(_R):
      m = _R * i + j
      b = j
      bp = (j - 1) % _R

      @pl.when(m >= 1)
      def _():
        store_wait(m - 1, bp)

      @pl.when(m + _R - 1 < _PER_W)
      def _():
        gather_start(m + _R - 1, bp)

      gather_wait(b)
      store_start(m, b)

  store_wait(_PER_W - 1, (_PER_W - 1) % _R)


@jax.jit
def _emb(idx, table):
  mesh = plsc.VectorSubcoreMesh(
      core_axis_name="c", subcore_axis_name="s",
      num_cores=_NC, num_subcores=_NS)
  f = functools.partial(
      pl.kernel,
      mesh=mesh,
      out_type=jax.ShapeDtypeStruct((_BATCH, _HIST, _EMB), jnp.float32),
      scratch_types=[
          pltpu.VMEM((_PER_W, _HIST), jnp.int32),
          pltpu.VMEM((_R, _HIST, _EMB), jnp.float32),
          [pltpu.SemaphoreType.DMA] * _R,
          [pltpu.SemaphoreType.DMA] * _R,
      ],
      compiler_params=pltpu.CompilerParams(use_tc_tiling_on_sc=False),
  )(_emb_body)
  return f(idx, table)


def kernel(data, W):
  idx = data.reshape(_NW, _PER_W, _HIST)
  return _emb(idx, W)
